# Initial kernel scaffold; baseline (speedup 1.0000x reference)
import functools

import jax
import jax.numpy as jnp
from jax import lax
from jax.experimental import pallas as pl
from jax.experimental.pallas import tpu as pltpu
from jax.experimental.pallas import tpu_sc as plsc

VOCAB = 100000
EMBED_DIM = 128
BATCH = 4096
HIST = 50

_INFO = plsc.get_sparse_core_info()
NC = _INFO.num_cores
NS = _INFO.num_subcores
NW = NC * NS

TOTAL = BATCH * HIST
CHUNK = 128
N_CHUNKS = TOTAL // CHUNK
CPW = N_CHUNKS // NW


def _gather_body(tokens_hbm, table_hbm, out_hbm, idx_v, buf, sem):
    wid = lax.axis_index("s") * NC + lax.axis_index("c")
    row0 = wid * CPW
    pltpu.sync_copy(tokens_hbm.at[pl.ds(row0, CPW)], idx_v)

    def step(j, carry):
        pltpu.async_copy(table_hbm.at[idx_v.at[j]], buf, sem).wait()
        pltpu.sync_copy(buf, out_hbm.at[pl.ds((row0 + j) * CHUNK, CHUNK)])
        return carry

    lax.fori_loop(0, CPW, step, 0)


@jax.jit
def kernel(tokens, parameters):
    idx = tokens.astype(jnp.int32).reshape(N_CHUNKS, CHUNK)
    mesh = plsc.VectorSubcoreMesh(core_axis_name="c", subcore_axis_name="s")
    out = pl.kernel(
        _gather_body,
        out_type=jax.ShapeDtypeStruct((TOTAL, EMBED_DIM), jnp.float32),
        mesh=mesh,
        scratch_types=[
            pltpu.VMEM((CPW, CHUNK), jnp.int32),
            pltpu.VMEM((CHUNK, EMBED_DIM), jnp.float32),
            pltpu.SemaphoreType.DMA,
        ],
    )(idx, parameters)
    return out.reshape(BATCH, HIST, EMBED_DIM)


# speedup vs baseline: 2.9732x; 2.9732x over previous
import functools

import jax
import jax.numpy as jnp
from jax import lax
from jax.experimental import pallas as pl
from jax.experimental.pallas import tpu as pltpu
from jax.experimental.pallas import tpu_sc as plsc

VOCAB = 100000
EMBED_DIM = 128
BATCH = 4096
HIST = 50

_INFO = plsc.get_sparse_core_info()
NC = _INFO.num_cores
NS = _INFO.num_subcores
NW = NC * NS

TOTAL = BATCH * HIST
CHUNK = 128
N_CHUNKS = TOTAL // CHUNK
CPW = N_CHUNKS // NW


def _gather_body(tokens_hbm, table_hbm, out_hbm, idx_v, buf, sem):
    wid = lax.axis_index("s") * NC + lax.axis_index("c")
    row0 = wid * CPW
    pltpu.sync_copy(tokens_hbm.at[wid], idx_v)

    def step(j, carry):
        pltpu.async_copy(table_hbm.at[idx_v.at[j]], buf, sem).wait()
        pltpu.sync_copy(buf, out_hbm.at[pl.ds((row0 + j) * CHUNK, CHUNK)])
        return carry

    lax.fori_loop(0, CPW, step, 0)


@jax.jit
def kernel(tokens, parameters):
    idx = tokens.astype(jnp.int32).reshape(NW, CPW, CHUNK)
    mesh = plsc.VectorSubcoreMesh(core_axis_name="c", subcore_axis_name="s")
    out = pl.kernel(
        _gather_body,
        out_type=jax.ShapeDtypeStruct((TOTAL, EMBED_DIM), jnp.float32),
        mesh=mesh,
        scratch_types=[
            pltpu.VMEM((CPW, CHUNK), jnp.int32),
            pltpu.VMEM((CHUNK, EMBED_DIM), jnp.float32),
            pltpu.SemaphoreType.DMA,
        ],
    )(idx, parameters)
    return out.reshape(BATCH, HIST, EMBED_DIM)


# speedup vs baseline: 3.3355x; 1.1218x over previous
import jax
import jax.numpy as jnp
from jax import lax
from jax.experimental import pallas as pl
from jax.experimental.pallas import tpu as pltpu
from jax.experimental.pallas import tpu_sc as plsc

VOCAB = 100000
EMBED_DIM = 128
BATCH = 4096
HIST = 50

_INFO = plsc.get_sparse_core_info()
NC = _INFO.num_cores
NS = _INFO.num_subcores
NW = NC * NS

TOTAL = BATCH * HIST
CHUNK = 128
N_CHUNKS = TOTAL // CHUNK
CPW = N_CHUNKS // NW
NBUF = 5


def _gather_body(tokens_hbm, table_hbm, out_hbm, idx_v, bufs, *sems):
    gsems = sems[:NBUF]
    wsems = sems[NBUF:]
    wid = lax.axis_index("s") * NC + lax.axis_index("c")
    row0 = wid * CPW
    pltpu.sync_copy(tokens_hbm.at[wid], idx_v)

    def g_start(j, b):
        pltpu.async_copy(table_hbm.at[idx_v.at[j]], bufs.at[b], gsems[b])

    def g_wait(j, b):
        pltpu.make_async_copy(table_hbm.at[idx_v.at[j]], bufs.at[b], gsems[b]).wait()

    def w_start(j, b):
        pltpu.async_copy(bufs.at[b], out_hbm.at[pl.ds((row0 + j) * CHUNK, CHUNK)], wsems[b])

    def w_wait(j, b):
        pltpu.make_async_copy(bufs.at[b], out_hbm.at[pl.ds((row0 + j) * CHUNK, CHUNK)], wsems[b]).wait()

    for b in range(NBUF - 1):
        g_start(b, b)

    def outer(t, carry):
        for b in range(NBUF):
            j = NBUF * t + b
            g_wait(j, b)
            w_start(j, b)
            bp = (b - 1) % NBUF
            if b == 0:
                @pl.when(t > 0)
                def _():
                    w_wait(j - 1, bp)
                    g_start(j + NBUF - 1, bp)

                @pl.when(t == 0)
                def _():
                    g_start(j + NBUF - 1, bp)
            else:
                w_wait(j - 1, bp)
                g_start(j + NBUF - 1, bp)
        return carry

    lax.fori_loop(0, CPW // NBUF - 1, outer, 0)

    t_last = CPW // NBUF - 1
    for b in range(NBUF):
        j = NBUF * t_last + b
        g_wait(j, b)
        w_start(j, b)
        bp = (b - 1) % NBUF
        w_wait(j - 1, bp)
        if b == 0:
            g_start(j + NBUF - 1, bp)
    w_wait(CPW - 1, (CPW - 1) % NBUF)


@jax.jit
def kernel(tokens, parameters):
    idx = tokens.astype(jnp.int32).reshape(NW, CPW, CHUNK)
    mesh = plsc.VectorSubcoreMesh(core_axis_name="c", subcore_axis_name="s")
    out = pl.kernel(
        _gather_body,
        out_type=jax.ShapeDtypeStruct((TOTAL, EMBED_DIM), jnp.float32),
        mesh=mesh,
        scratch_types=[
            pltpu.VMEM((CPW, CHUNK), jnp.int32),
            pltpu.VMEM((NBUF, CHUNK, EMBED_DIM), jnp.float32),
        ] + [pltpu.SemaphoreType.DMA] * (2 * NBUF),
    )(idx, parameters)
    return out.reshape(BATCH, HIST, EMBED_DIM)


# speedup vs baseline: 5.9761x; 1.7917x over previous
import jax
import jax.numpy as jnp
from jax import lax
from jax.experimental import pallas as pl
from jax.experimental.pallas import tpu as pltpu
from jax.experimental.pallas import tpu_sc as plsc

VOCAB = 100000
EMBED_DIM = 128
BATCH = 4096
HIST = 50

_INFO = plsc.get_sparse_core_info()
NC = _INFO.num_cores
NS = _INFO.num_subcores
NW = NC * NS

BPW = BATCH // NW
NBUF = 8


def _gather_body(tokens_hbm, table_hbm, out_hbm, idx_v, bufs, *sems):
    gsems = sems[:NBUF]
    wsems = sems[NBUF:]
    wid = lax.axis_index("s") * NC + lax.axis_index("c")
    b0 = wid * BPW
    pltpu.sync_copy(tokens_hbm.at[pl.ds(b0, BPW)], idx_v)

    def g_start(j, b):
        pltpu.async_copy(table_hbm.at[idx_v.at[j]], bufs.at[b], gsems[b])

    def g_wait(j, b):
        pltpu.make_async_copy(table_hbm.at[idx_v.at[j]], bufs.at[b], gsems[b]).wait()

    def w_start(j, b):
        pltpu.async_copy(bufs.at[b], out_hbm.at[b0 + j], wsems[b])

    def w_wait(j, b):
        pltpu.make_async_copy(bufs.at[b], out_hbm.at[b0 + j], wsems[b]).wait()

    for b in range(NBUF - 1):
        g_start(b, b)

    def outer(t, carry):
        for b in range(NBUF):
            j = NBUF * t + b
            bp = (b - 1) % NBUF
            g_wait(j, b)
            w_start(j, b)
            if b == 0:
                @pl.when(t > 0)
                def _():
                    w_wait(j - 1, bp)
                    g_start(j + NBUF - 1, bp)

                @pl.when(t == 0)
                def _():
                    g_start(j + NBUF - 1, bp)
            else:
                w_wait(j - 1, bp)
                g_start(j + NBUF - 1, bp)
        return carry

    lax.fori_loop(0, BPW // NBUF - 1, outer, 0)

    t_last = BPW // NBUF - 1
    for b in range(NBUF):
        j = NBUF * t_last + b
        bp = (b - 1) % NBUF
        g_wait(j, b)
        w_start(j, b)
        w_wait(j - 1, bp)
        if b == 0:
            g_start(j + NBUF - 1, bp)
    w_wait(BPW - 1, (BPW - 1) % NBUF)


@jax.jit
def kernel(tokens, parameters):
    mesh = plsc.VectorSubcoreMesh(core_axis_name="c", subcore_axis_name="s")
    return pl.kernel(
        _gather_body,
        out_type=jax.ShapeDtypeStruct((BATCH, HIST, EMBED_DIM), jnp.float32),
        mesh=mesh,
        scratch_types=[
            pltpu.VMEM((BPW, HIST), jnp.int32),
            pltpu.VMEM((NBUF, HIST, EMBED_DIM), jnp.float32),
        ] + [pltpu.SemaphoreType.DMA] * (2 * NBUF),
    )(tokens.astype(jnp.int32), parameters)


# speedup vs baseline: 10.7504x; 1.7989x over previous
import jax
import jax.numpy as jnp
from jax import lax
from jax.experimental import pallas as pl
from jax.experimental.pallas import tpu as pltpu
from jax.experimental.pallas import tpu_sc as plsc

VOCAB = 100000
EMBED_DIM = 128
BATCH = 4096
HIST = 50

_INFO = plsc.get_sparse_core_info()
NC = _INFO.num_cores
NS = _INFO.num_subcores
NW = NC * NS

BPW = BATCH // NW
CPW = HIST
NBUF = 5


def _gather_body(tokens_hbm, table_hbm, out_hbm, idx_v, bufs, *sems):
    gsems = sems[:NBUF]
    wsems = sems[NBUF:]
    wid = lax.axis_index("s") * NC + lax.axis_index("c")
    c0 = wid * BPW
    pltpu.sync_copy(tokens_hbm.at[:, pl.ds(c0, BPW)], idx_v)

    def g_start(j, b):
        pltpu.async_copy(table_hbm.at[idx_v.at[j]], bufs.at[b], gsems[b])

    def g_wait(j, b):
        pltpu.make_async_copy(table_hbm.at[idx_v.at[j]], bufs.at[b], gsems[b]).wait()

    def w_start(j, b):
        pltpu.async_copy(bufs.at[b], out_hbm.at[j, pl.ds(c0, BPW)], wsems[b])

    def w_wait(j, b):
        pltpu.make_async_copy(bufs.at[b], out_hbm.at[j, pl.ds(c0, BPW)], wsems[b]).wait()

    for b in range(NBUF - 1):
        g_start(b, b)

    def outer(t, carry):
        for b in range(NBUF):
            j = NBUF * t + b
            bp = (b - 1) % NBUF
            g_wait(j, b)
            w_start(j, b)
            if b == 0:
                @pl.when(t > 0)
                def _():
                    w_wait(j - 1, bp)
                    g_start(j + NBUF - 1, bp)

                @pl.when(t == 0)
                def _():
                    g_start(j + NBUF - 1, bp)
            else:
                w_wait(j - 1, bp)
                g_start(j + NBUF - 1, bp)
        return carry

    lax.fori_loop(0, CPW // NBUF - 1, outer, 0)

    t_last = CPW // NBUF - 1
    for b in range(NBUF):
        j = NBUF * t_last + b
        bp = (b - 1) % NBUF
        g_wait(j, b)
        w_start(j, b)
        w_wait(j - 1, bp)
        if b == 0:
            g_start(j + NBUF - 1, bp)
    w_wait(CPW - 1, (CPW - 1) % NBUF)


@jax.jit
def kernel(tokens, parameters):
    mesh = plsc.VectorSubcoreMesh(core_axis_name="c", subcore_axis_name="s")
    out_t = pl.kernel(
        _gather_body,
        out_type=jax.ShapeDtypeStruct((HIST, BATCH, EMBED_DIM), jnp.float32),
        mesh=mesh,
        scratch_types=[
            pltpu.VMEM((HIST, BPW), jnp.int32),
            pltpu.VMEM((NBUF, BPW, EMBED_DIM), jnp.float32),
        ] + [pltpu.SemaphoreType.DMA] * (2 * NBUF),
    )(tokens.astype(jnp.int32).T, parameters)
    return out_t.transpose(1, 0, 2)


# speedup vs baseline: 10.8599x; 1.0102x over previous
import jax
import jax.numpy as jnp
from jax import lax
from jax.experimental import pallas as pl
from jax.experimental.pallas import tpu as pltpu
from jax.experimental.pallas import tpu_sc as plsc

VOCAB = 100000
EMBED_DIM = 128
BATCH = 4096
HIST = 50

_INFO = plsc.get_sparse_core_info()
NC = _INFO.num_cores
NS = _INFO.num_subcores
NW = NC * NS

BPW = BATCH // NW
CPW = HIST
NBUF = 7


def _gather_body(tokens_hbm, table_hbm, out_hbm, idx_v, bufs, *sems):
    gsems = sems[:NBUF]
    wsems = sems[NBUF:]
    wid = lax.axis_index("s") * NC + lax.axis_index("c")
    c0 = wid * BPW
    pltpu.sync_copy(tokens_hbm.at[:, pl.ds(c0, BPW)], idx_v)

    def g_start(j, b):
        pltpu.async_copy(table_hbm.at[idx_v.at[j]], bufs.at[b], gsems[b])

    def g_wait(j, b):
        pltpu.make_async_copy(table_hbm.at[idx_v.at[j]], bufs.at[b], gsems[b]).wait()

    def w_start(j, b):
        pltpu.async_copy(bufs.at[b], out_hbm.at[j, pl.ds(c0, BPW)], wsems[b])

    def w_wait(j, b):
        pltpu.make_async_copy(bufs.at[b], out_hbm.at[j, pl.ds(c0, BPW)], wsems[b]).wait()

    for b in range(NBUF - 1):
        g_start(b, b)

    def chunk_step(t, b):
        j = NBUF * t + b
        bp = (b - 1) % NBUF
        g_wait(j, b)
        w_start(j, b)

        def advance():
            w_wait(j - 1, bp)
            if isinstance(j, int):
                if j + NBUF - 1 < CPW:
                    g_start(j + NBUF - 1, bp)
            else:
                @pl.when(j + NBUF - 1 < CPW)
                def _():
                    g_start(j + NBUF - 1, bp)

        if b == 0 and isinstance(t, int):
            advance()
        elif b == 0:
            @pl.when(t > 0)
            def _():
                advance()

            @pl.when(t == 0)
            def _():
                g_start(j + NBUF - 1, bp)
        else:
            advance()

    def outer(t, carry):
        for b in range(NBUF):
            chunk_step(t, b)
        return carry

    n_full = CPW // NBUF
    lax.fori_loop(0, n_full, outer, 0)
    for b in range(CPW % NBUF):
        chunk_step(n_full, b)
    w_wait(CPW - 1, (CPW - 1) % NBUF)


@jax.jit
def kernel(tokens, parameters):
    mesh = plsc.VectorSubcoreMesh(core_axis_name="c", subcore_axis_name="s")
    out_t = pl.kernel(
        _gather_body,
        out_type=jax.ShapeDtypeStruct((HIST, BATCH, EMBED_DIM), jnp.float32),
        mesh=mesh,
        scratch_types=[
            pltpu.VMEM((HIST, BPW), jnp.int32),
            pltpu.VMEM((NBUF, BPW, EMBED_DIM), jnp.float32),
        ] + [pltpu.SemaphoreType.DMA] * (2 * NBUF),
    )(tokens.astype(jnp.int32).T, parameters)
    return out_t.transpose(1, 0, 2)
